# baseline (speedup 1.0000x reference)
import functools

import jax
import jax.numpy as jnp
from jax import lax
from jax.experimental import pallas as pl
from jax.experimental.pallas import tpu as pltpu
from jax.experimental.pallas import tpu_sc as plsc

_N = 16384
_DP = 1024
_NV = _DP // 16
_K = 32
_TEMP = 2.0
_NW = 32
_TC_ROWS = 9728
_TC_BLOCK = 512
_SC_ROWS = _N - _TC_ROWS


def _tc_kernel(x_ref, out_ref):
    x = x_ref[...]
    r, dp = x.shape
    inv_t = jnp.float32(1.0 / _TEMP)
    m = jnp.max(x, axis=1, keepdims=True)
    s = jnp.sum(jnp.exp((x - m) * inv_t), axis=1, keepdims=True)
    col = jax.lax.broadcasted_iota(jnp.int32, (r, dp), 1)
    neginf = jnp.float32(-jnp.inf)

    def body(_, carry):
        xc, out, denom = carry
        m1 = jnp.max(xc, axis=1, keepdims=True)
        idx = jnp.min(jnp.where(xc == m1, col, dp), axis=1, keepdims=True)
        sel = col == idx
        ev = jnp.exp((m1 - m) * inv_t)
        out = jnp.where(sel, ev / denom, out)
        xc = jnp.where(sel, neginf, xc)
        return xc, out, denom - ev

    _, out, _ = jax.lax.fori_loop(
        0, _K, body, (x, jnp.zeros_like(x), s), unroll=True
    )
    out_ref[...] = out


def _tc_latents(xp):
    return pl.pallas_call(
        _tc_kernel,
        grid=(_TC_ROWS // _TC_BLOCK,),
        in_specs=[pl.BlockSpec((_TC_BLOCK, _DP), lambda i: (i, 0))],
        out_specs=pl.BlockSpec((_TC_BLOCK, _DP), lambda i: (i, 0)),
        out_shape=jax.ShapeDtypeStruct((_TC_ROWS, _DP), jnp.float32),
    )(xp)


_RPW = _SC_ROWS // _NW


def _sc_body(x_hbm, out_hbm, xbuf0, xbuf1, ebuf, outbuf, insem0, insem1):
    xbufs = (xbuf0, xbuf1)
    insems = (insem0, insem1)
    f32 = jnp.float32
    wid = lax.axis_index("s") * 2 + lax.axis_index("c")
    base = _TC_ROWS + wid * _RPW
    iota = lax.iota(jnp.int32, 16)
    zero_v = jnp.zeros((16,), f32)

    for j in range(_NV):
        outbuf[pl.ds(j * 16, 16)] = zero_v

    pltpu.async_copy(x_hbm.at[base], xbuf0, insem0)

    def row_body(g, _):
        for b in range(2):
            row = base + 2 * g + b
            pltpu.make_async_copy(x_hbm.at[row], xbufs[b], insems[b]).wait()

            @pl.when(row + 1 < base + _RPW)
            def _():
                pltpu.async_copy(
                    x_hbm.at[row + 1], xbufs[1 - b], insems[1 - b]
                )

            xb = xbufs[b]
            mv = xb[pl.ds(0, 16)]
            for j in range(1, _NV):
                mv = jnp.maximum(mv, xb[pl.ds(j * 16, 16)])
            m = jnp.max(mv)
            ms = jnp.broadcast_to(m, (16,))
            acc = zero_v
            cmax = zero_v
            for j in range(_NV):
                v = xb[pl.ds(j * 16, 16)]
                e = jnp.exp((v - ms) * f32(0.5))
                ebuf[pl.ds(j * 16, 16)] = e
                acc = acc + e
                cmax = jnp.maximum(cmax, e)
            s_tot = jnp.sum(acc)

            vals = [zero_v, zero_v]
            idxs = [jnp.zeros((16,), jnp.int32), jnp.zeros((16,), jnp.int32)]
            for k in range(_K):
                mx = jnp.max(cmax)
                mx_s = jnp.broadcast_to(mx, (16,))
                lane = plsc.all_reduce_ffs(cmax == mx_s)
                gs = []
                cands = []
                for t in range(4):
                    gidx = (iota + 16 * t) * 16 + lane
                    gt = plsc.load_gather(ebuf, [gidx])
                    gs.append(gt)
                    ft = plsc.all_reduce_ffs(gt == mx_s)
                    cands.append(jnp.where(ft < 16, ft + 16 * t, 9999))
                jv = jnp.minimum(
                    jnp.minimum(cands[0], cands[1]),
                    jnp.minimum(cands[2], cands[3]),
                )
                w = jv * 16 + lane
                plsc.store_scatter(ebuf, [w], zero_v, mask=iota == 0)
                nmv = zero_v
                for t in range(4):
                    hit = (iota + 16 * t) == jv
                    nmv = jnp.maximum(nmv, jnp.where(hit, f32(0.0), gs[t]))
                nm = jnp.max(nmv)
                h = k // 16
                vals[h] = jnp.where(iota == (k % 16), mx_s, vals[h])
                idxs[h] = jnp.where(iota == (k % 16), w, idxs[h])
                cmax = jnp.where(iota == lane, jnp.broadcast_to(nm, (16,)), cmax)

            s_s = jnp.broadcast_to(s_tot, (16,))
            c_lo = plsc.cumsum(vals[0])
            d_lo = s_s - (c_lo - vals[0])
            lo_tot = jnp.sum(vals[0])
            c_hi = plsc.cumsum(vals[1])
            d_hi = s_s - jnp.broadcast_to(lo_tot, (16,)) - (c_hi - vals[1])
            o_lo = vals[0] / d_lo
            o_hi = vals[1] / d_hi
            plsc.store_scatter(outbuf, [idxs[0]], o_lo)
            plsc.store_scatter(outbuf, [idxs[1]], o_hi)
            pltpu.sync_copy(outbuf, out_hbm.at[row - _TC_ROWS])
            plsc.store_scatter(outbuf, [idxs[0]], zero_v)
            plsc.store_scatter(outbuf, [idxs[1]], zero_v)
        return 0

    lax.fori_loop(0, _RPW // 2, row_body, 0)


def _sc_latents(xp):
    mesh = plsc.VectorSubcoreMesh(
        core_axis_name="c", subcore_axis_name="s", num_cores=2, num_subcores=16
    )
    run = functools.partial(
        pl.kernel,
        out_type=jax.ShapeDtypeStruct((_SC_ROWS, _DP), jnp.float32),
        mesh=mesh,
        compiler_params=pltpu.CompilerParams(needs_layout_passes=False),
        scratch_types=[
            pltpu.VMEM((_DP,), jnp.float32),
            pltpu.VMEM((_DP,), jnp.float32),
            pltpu.VMEM((_DP,), jnp.float32),
            pltpu.VMEM((_DP,), jnp.float32),
            pltpu.SemaphoreType.DMA,
            pltpu.SemaphoreType.DMA,
        ],
    )(_sc_body)
    return run(xp)


def kernel(normu, cls):
    n, d = cls.shape
    xp = jnp.pad(cls, ((0, 0), (0, _DP - d)), constant_values=-1e30)
    out_tc = _tc_latents(xp)
    out_sc = _sc_latents(xp)
    out = jnp.concatenate([out_tc, out_sc], axis=0)
    return (normu, out[:, :d])


# speedup vs baseline: 1.0684x; 1.0684x over previous
import functools

import jax
import jax.numpy as jnp
from jax import lax
from jax.experimental import pallas as pl
from jax.experimental.pallas import tpu as pltpu
from jax.experimental.pallas import tpu_sc as plsc

_N = 16384
_DP = 1024
_NV = _DP // 16
_K = 32
_TEMP = 2.0
_NW = 32
_TC_ROWS = 10240
_TC_BLOCK = 512
_SC_ROWS = _N - _TC_ROWS


def _tc_kernel(x_ref, out_ref):
    x = x_ref[...]
    r, dp = x.shape
    inv_t = jnp.float32(1.0 / _TEMP)
    m = jnp.max(x, axis=1, keepdims=True)
    s = jnp.sum(jnp.exp((x - m) * inv_t), axis=1, keepdims=True)
    col = jax.lax.broadcasted_iota(jnp.int32, (r, dp), 1)
    neginf = jnp.float32(-jnp.inf)

    def body(_, carry):
        xc, out, denom = carry
        m1 = jnp.max(xc, axis=1, keepdims=True)
        idx = jnp.min(jnp.where(xc == m1, col, dp), axis=1, keepdims=True)
        sel = col == idx
        ev = jnp.exp((m1 - m) * inv_t)
        out = jnp.where(sel, ev / denom, out)
        xc = jnp.where(sel, neginf, xc)
        return xc, out, denom - ev

    _, out, _ = jax.lax.fori_loop(
        0, _K, body, (x, jnp.zeros_like(x), s), unroll=True
    )
    out_ref[...] = out


def _tc_latents(xp):
    return pl.pallas_call(
        _tc_kernel,
        grid=(_TC_ROWS // _TC_BLOCK,),
        in_specs=[pl.BlockSpec((_TC_BLOCK, _DP), lambda i: (i, 0))],
        out_specs=pl.BlockSpec((_TC_BLOCK, _DP), lambda i: (i, 0)),
        out_shape=jax.ShapeDtypeStruct((_TC_ROWS, _DP), jnp.float32),
    )(xp)


_RPW = _SC_ROWS // _NW


def _sc_body(x_hbm, out_hbm, xbuf0, xbuf1, ebuf, outbuf, insem0, insem1):
    xbufs = (xbuf0, xbuf1)
    insems = (insem0, insem1)
    f32 = jnp.float32
    wid = lax.axis_index("s") * 2 + lax.axis_index("c")
    base = _TC_ROWS + wid * _RPW
    iota = lax.iota(jnp.int32, 16)
    zero_v = jnp.zeros((16,), f32)

    for j in range(_NV):
        outbuf[pl.ds(j * 16, 16)] = zero_v

    pltpu.async_copy(x_hbm.at[base], xbuf0, insem0)

    def row_body(g, _):
        for b in range(2):
            row = base + 2 * g + b
            pltpu.make_async_copy(x_hbm.at[row], xbufs[b], insems[b]).wait()

            @pl.when(row + 1 < base + _RPW)
            def _():
                pltpu.async_copy(
                    x_hbm.at[row + 1], xbufs[1 - b], insems[1 - b]
                )

            xb = xbufs[b]
            mv = xb[pl.ds(0, 16)]
            for j in range(1, _NV):
                mv = jnp.maximum(mv, xb[pl.ds(j * 16, 16)])
            m = jnp.max(mv)
            ms = jnp.broadcast_to(m, (16,))
            acc = zero_v
            cmax = zero_v
            for j in range(_NV):
                v = xb[pl.ds(j * 16, 16)]
                e = jnp.exp((v - ms) * f32(0.5))
                ebuf[pl.ds(j * 16, 16)] = e
                acc = acc + e
                cmax = jnp.maximum(cmax, e)
            s_tot = jnp.sum(acc)

            vals = [zero_v, zero_v]
            idxs = [jnp.zeros((16,), jnp.int32), jnp.zeros((16,), jnp.int32)]
            for k in range(_K):
                mx = jnp.max(cmax)
                mx_s = jnp.broadcast_to(mx, (16,))
                lane = plsc.all_reduce_ffs(cmax == mx_s)
                gs = []
                cands = []
                for t in range(4):
                    gidx = (iota + 16 * t) * 16 + lane
                    gt = plsc.load_gather(ebuf, [gidx])
                    gs.append(gt)
                    ft = plsc.all_reduce_ffs(gt == mx_s)
                    cands.append(jnp.where(ft < 16, ft + 16 * t, 9999))
                jv = jnp.minimum(
                    jnp.minimum(cands[0], cands[1]),
                    jnp.minimum(cands[2], cands[3]),
                )
                w = jv * 16 + lane
                plsc.store_scatter(ebuf, [w], zero_v, mask=iota == 0)
                nmv = zero_v
                for t in range(4):
                    hit = (iota + 16 * t) == jv
                    nmv = jnp.maximum(nmv, jnp.where(hit, f32(0.0), gs[t]))
                nm = jnp.max(nmv)
                h = k // 16
                vals[h] = jnp.where(iota == (k % 16), mx_s, vals[h])
                idxs[h] = jnp.where(iota == (k % 16), w, idxs[h])
                cmax = jnp.where(iota == lane, jnp.broadcast_to(nm, (16,)), cmax)

            s_s = jnp.broadcast_to(s_tot, (16,))
            c_lo = plsc.cumsum(vals[0])
            d_lo = s_s - (c_lo - vals[0])
            lo_tot = jnp.sum(vals[0])
            c_hi = plsc.cumsum(vals[1])
            d_hi = s_s - jnp.broadcast_to(lo_tot, (16,)) - (c_hi - vals[1])
            o_lo = vals[0] / d_lo
            o_hi = vals[1] / d_hi
            plsc.store_scatter(outbuf, [idxs[0]], o_lo)
            plsc.store_scatter(outbuf, [idxs[1]], o_hi)
            pltpu.sync_copy(outbuf, out_hbm.at[row - _TC_ROWS])
            plsc.store_scatter(outbuf, [idxs[0]], zero_v)
            plsc.store_scatter(outbuf, [idxs[1]], zero_v)
        return 0

    lax.fori_loop(0, _RPW // 2, row_body, 0)


def _sc_latents(xp):
    mesh = plsc.VectorSubcoreMesh(
        core_axis_name="c", subcore_axis_name="s", num_cores=2, num_subcores=16
    )
    run = functools.partial(
        pl.kernel,
        out_type=jax.ShapeDtypeStruct((_SC_ROWS, _DP), jnp.float32),
        mesh=mesh,
        compiler_params=pltpu.CompilerParams(needs_layout_passes=False),
        scratch_types=[
            pltpu.VMEM((_DP,), jnp.float32),
            pltpu.VMEM((_DP,), jnp.float32),
            pltpu.VMEM((_DP,), jnp.float32),
            pltpu.VMEM((_DP,), jnp.float32),
            pltpu.SemaphoreType.DMA,
            pltpu.SemaphoreType.DMA,
        ],
    )(_sc_body)
    return run(xp)


def kernel(normu, cls):
    n, d = cls.shape
    xp = jnp.pad(cls, ((0, 0), (0, _DP - d)), constant_values=-1e30)
    out_tc = _tc_latents(xp)
    out_sc = _sc_latents(xp)
    out = jnp.concatenate([out_tc, out_sc], axis=0)
    return (normu, out[:, :d])
